# baseline (speedup 1.0000x reference)
import functools

import jax
import jax.numpy as jnp
from jax import lax
from jax.experimental import pallas as pl
from jax.experimental.pallas import tpu as pltpu
from jax.experimental.pallas import tpu_sc as plsc

_B = 16
_H = 32
_W = 32
_HW = _H * _W
_C = 256
_NC = 2
_NS = 16
_NW = _NC * _NS
_RPW = (2 * _C) // _NW
_L = 16
_TAB = 40


def _sc_body(row_hbm, col_hbm, pose_hbm, out_m, out_p, tab_v, buf_v, pbuf_v,
             ptab_v, sem):
    cid = lax.axis_index("c")
    sid = lax.axis_index("s")
    wid = sid * _NC + cid
    iota = jax.lax.iota(jnp.int32, _L)
    is_col = wid < _NS * _NC // 2

    @pl.when(is_col)
    def _col_rows():
        pltpu.sync_copy(col_hbm.at[pl.ds(0, _TAB)], tab_v)
        for r in range(_RPW):
            c = wid * _RPW + r
            cvec = jnp.full((_L,), 0, jnp.int32) + c
            lo = plsc.load_gather(tab_v, [iota + 1, cvec])
            hi = plsc.load_gather(tab_v, [iota + 1 + _L, cvec])

            def _tile(t, carry, r=r, lo=lo, hi=hi):
                buf_v[r, pl.ds(t * _W, _L)] = lo
                buf_v[r, pl.ds(t * _W + _L, _L)] = hi
                return carry

            lax.fori_loop(0, _H, _tile, 0)

    @pl.when(jnp.logical_not(is_col))
    def _row_rows():
        pltpu.sync_copy(row_hbm.at[pl.ds(0, _TAB)], tab_v)
        for r in range(_RPW):
            cr = wid * _RPW + r - _C

            def _span(y, carry, r=r, cr=cr):
                yvec = jnp.full((_L,), 1, jnp.int32) + y
                crvec = jnp.full((_L,), 0, jnp.int32) + cr
                vec = plsc.load_gather(tab_v, [yvec, crvec])
                buf_v[r, pl.ds(y * _W, _L)] = vec
                buf_v[r, pl.ds(y * _W + _L, _L)] = vec
                return carry

            lax.fori_loop(0, _H, _span, 0)

    copies = [
        pltpu.async_copy(buf_v, out_m.at[b, pl.ds(wid * _RPW, _RPW)], sem)
        for b in range(_B)
    ]
    for cp in copies:
        cp.wait()

    @pl.when(wid == 0)
    def _pose():
        pltpu.sync_copy(pose_hbm.at[pl.ds(0, 8)], ptab_v)
        for k in range(_C // _L):
            v = ptab_v[0, pl.ds(k * _L, _L)]
            pbuf_v[pl.ds(k * _L, _L)] = v
            pbuf_v[pl.ds(_C + k * _L, _L)] = v
        for b in range(_B):
            pltpu.sync_copy(pbuf_v, out_p.at[pl.ds(b * 2 * _C, 2 * _C)])


@jax.jit
def _sc_call(row_embed, col_embed, pose_token_embed):
    mesh = plsc.VectorSubcoreMesh(core_axis_name="c", subcore_axis_name="s")
    run = pl.kernel(
        _sc_body,
        out_type=[
            jax.ShapeDtypeStruct((_B, 2 * _C, _HW), jnp.float32),
            jax.ShapeDtypeStruct((_B * 2 * _C,), jnp.float32),
        ],
        mesh=mesh,
        scratch_types=[
            pltpu.VMEM((_TAB, _C), jnp.float32),
            pltpu.VMEM((_RPW, _HW), jnp.float32),
            pltpu.VMEM((2 * _C,), jnp.float32),
            pltpu.VMEM((8, _C), jnp.float32),
            pltpu.SemaphoreType.DMA,
        ],
        compiler_params=pltpu.CompilerParams(
            use_tc_tiling_on_sc=False, needs_layout_passes=False
        ),
    )
    return run(row_embed, col_embed, pose_token_embed)


def kernel(x, row_embed, col_embed, pose_token_embed):
    m_flat, p_flat = _sc_call(row_embed, col_embed, pose_token_embed)
    b = x.shape[0]
    h, w = x.shape[-2], x.shape[-1]
    return jnp.reshape(p_flat, (b, 2 * _C)), jnp.reshape(m_flat, (b, 2 * _C, h, w))
